# baseline (speedup 1.0000x reference)
import functools

import jax
import jax.numpy as jnp
from jax import lax
from jax.experimental import pallas as pl
from jax.experimental.pallas import tpu as pltpu
from jax.experimental.pallas import tpu_sc as plsc

GAMMA = 12.0
HIDDEN = 128
LANES = 16
VPR = HIDDEN // LANES
CHUNK = 64
NBUF = 5


def kernel(entity_emb, relation_emb, head_ids, rel_ids, tail_ids):
    B = head_ids.shape[0]
    info = plsc.get_sparse_core_info()
    NC, NS = info.num_cores, info.num_subcores
    NW = NC * NS
    per_w = B // NW
    n_chunks = per_w // CHUNK

    mesh = plsc.VectorSubcoreMesh(core_axis_name="c", subcore_axis_name="s")

    @functools.partial(
        pl.kernel,
        mesh=mesh,
        out_type=jax.ShapeDtypeStruct((B,), jnp.float32),
        scratch_types=[
            pltpu.VMEM((per_w,), jnp.int32),
            pltpu.VMEM((per_w,), jnp.int32),
            pltpu.VMEM((per_w,), jnp.int32),
            pltpu.VMEM((NBUF * CHUNK, HIDDEN), jnp.float32),
            pltpu.VMEM((NBUF * CHUNK, HIDDEN), jnp.float32),
            pltpu.VMEM((NBUF * CHUNK, HIDDEN), jnp.float32),
            pltpu.VMEM((per_w,), jnp.float32),
            pltpu.VMEM((LANES * LANES,), jnp.float32),
            pltpu.SemaphoreType.DMA((NBUF + 2,)),
        ],
        compiler_params=pltpu.CompilerParams(needs_layout_passes=False),
    )
    def tec_kernel(ent_hbm, rel_hbm, hid_hbm, rid_hbm, tid_hbm, out_hbm,
                   hi_v, ti_v, ri_v, h_v, t_v, r_v, out_v, ts_v, semarr):
        wid = lax.axis_index("s") * NC + lax.axis_index("c")
        wbase = wid * per_w
        lanes = lax.iota(jnp.int32, LANES)

        wsl = pl.ds(wbase, per_w)
        id_descs = (
            pltpu.make_async_copy(hid_hbm.at[wsl], hi_v, semarr.at[NBUF]),
            pltpu.make_async_copy(tid_hbm.at[wsl], ti_v, semarr.at[NBUF]),
            pltpu.make_async_copy(rid_hbm.at[wsl], ri_v, semarr.at[NBUF]),
        )
        for d in id_descs:
            d.start()
        for d in id_descs:
            d.wait()

        def gather_descs(c, bo, sem):
            sl = pl.ds(c * CHUNK, CHUNK)
            dst = pl.ds(bo, CHUNK)
            return (
                pltpu.make_async_copy(ent_hbm.at[hi_v.at[sl]], h_v.at[dst], sem),
                pltpu.make_async_copy(ent_hbm.at[ti_v.at[sl]], t_v.at[dst], sem),
                pltpu.make_async_copy(rel_hbm.at[ri_v.at[sl]], r_v.at[dst], sem),
            )

        for b in range(NBUF):
            for d in gather_descs(b, b * CHUNK, semarr.at[b]):
                d.start()

        def compute_chunk(c, bo):
            def body(g, carry):
                base = g * LANES
                ilv = 4
                for tp in range(0, LANES, ilv):
                    accs = [jnp.zeros((LANES,), jnp.float32)] * ilv
                    for j in range(VPR):
                        for k in range(ilv):
                            i = bo + base + tp + k
                            hv = h_v[i, pl.ds(j * LANES, LANES)]
                            rv = r_v[i, pl.ds(j * LANES, LANES)]
                            tv = t_v[i, pl.ds(j * LANES, LANES)]
                            accs[k] = accs[k] + jnp.abs(hv + rv - tv)
                    for k in range(ilv):
                        col = jnp.full((LANES,), tp + k, jnp.int32)
                        plsc.store_scatter(ts_v, [lanes * LANES + col], accs[k])
                vals = [ts_v[pl.ds(l * LANES, LANES)] for l in range(LANES)]
                while len(vals) > 1:
                    vals = [a + b for a, b in zip(vals[::2], vals[1::2])]
                tot = vals[0]
                score = jnp.full((LANES,), GAMMA, jnp.float32) - tot
                obase = jnp.full((LANES,), c * CHUNK, jnp.int32) + base
                plsc.store_scatter(out_v, [obase + lanes], score)
                return carry

            lax.fori_loop(0, CHUNK // LANES, body, 0)

        def out_desc(c):
            return pltpu.make_async_copy(
                out_v.at[pl.ds(c * CHUNK, CHUNK)],
                out_hbm.at[pl.ds(wbase + c * CHUNK, CHUNK)],
                semarr.at[NBUF + 1],
            )

        def chunk_body(c, carry):
            par = lax.rem(c, NBUF)
            bo = par * CHUNK
            sem = semarr.at[par]
            for d in gather_descs(c, bo, sem):
                d.wait()
            compute_chunk(c, bo)

            @pl.when(c + NBUF < n_chunks)
            def _():
                for d in gather_descs(c + NBUF, bo, sem):
                    d.start()

            out_desc(c).start()
            return carry

        lax.fori_loop(0, n_chunks, chunk_body, 0)

        def drain_body(c, carry):
            out_desc(c).wait()
            return carry

        lax.fori_loop(0, n_chunks, drain_body, 0)

    return tec_kernel(entity_emb, relation_emb, head_ids, rel_ids, tail_ids)


# speedup vs baseline: 1.0188x; 1.0188x over previous
import functools

import jax
import jax.numpy as jnp
from jax import lax
from jax.experimental import pallas as pl
from jax.experimental.pallas import tpu as pltpu
from jax.experimental.pallas import tpu_sc as plsc

GAMMA = 12.0
HIDDEN = 128
LANES = 16
VPR = HIDDEN // LANES
CHUNK = 64
NBUF = 4


def kernel(entity_emb, relation_emb, head_ids, rel_ids, tail_ids):
    B = head_ids.shape[0]
    info = plsc.get_sparse_core_info()
    NC, NS = info.num_cores, info.num_subcores
    NW = NC * NS
    per_w = B // NW
    n_chunks = per_w // CHUNK

    mesh = plsc.VectorSubcoreMesh(core_axis_name="c", subcore_axis_name="s")

    @functools.partial(
        pl.kernel,
        mesh=mesh,
        out_type=jax.ShapeDtypeStruct((B,), jnp.float32),
        scratch_types=[
            pltpu.VMEM((per_w,), jnp.int32),
            pltpu.VMEM((per_w,), jnp.int32),
            pltpu.VMEM((per_w,), jnp.int32),
            pltpu.VMEM((NBUF * CHUNK, HIDDEN), jnp.float32),
            pltpu.VMEM((NBUF * CHUNK, HIDDEN), jnp.float32),
            pltpu.VMEM((NBUF * CHUNK, HIDDEN), jnp.float32),
            pltpu.VMEM((per_w,), jnp.float32),
            pltpu.VMEM((LANES * LANES,), jnp.float32),
            pltpu.SemaphoreType.DMA((NBUF + 2,)),
        ],
        compiler_params=pltpu.CompilerParams(needs_layout_passes=False),
    )
    def tec_kernel(ent_hbm, rel_hbm, hid_hbm, rid_hbm, tid_hbm, out_hbm,
                   hi_v, ti_v, ri_v, h_v, t_v, r_v, out_v, ts_v, semarr):
        wid = lax.axis_index("s") * NC + lax.axis_index("c")
        wbase = wid * per_w
        lanes = lax.iota(jnp.int32, LANES)

        wsl = pl.ds(wbase, per_w)
        id_descs = (
            pltpu.make_async_copy(hid_hbm.at[wsl], hi_v, semarr.at[NBUF]),
            pltpu.make_async_copy(tid_hbm.at[wsl], ti_v, semarr.at[NBUF]),
            pltpu.make_async_copy(rid_hbm.at[wsl], ri_v, semarr.at[NBUF]),
        )
        for d in id_descs:
            d.start()
        for d in id_descs:
            d.wait()

        def gather_descs(c, bo, sem):
            sl = pl.ds(c * CHUNK, CHUNK)
            dst = pl.ds(bo, CHUNK)
            return (
                pltpu.make_async_copy(ent_hbm.at[hi_v.at[sl]], h_v.at[dst], sem),
                pltpu.make_async_copy(ent_hbm.at[ti_v.at[sl]], t_v.at[dst], sem),
                pltpu.make_async_copy(rel_hbm.at[ri_v.at[sl]], r_v.at[dst], sem),
            )

        for b in range(NBUF):
            for d in gather_descs(b, b * CHUNK, semarr.at[b]):
                d.start()

        def compute_chunk(c, bo):
            def body(g, carry):
                base = g * LANES
                ilv = 4
                for tp in range(0, LANES, ilv):
                    accs = [jnp.zeros((LANES,), jnp.float32)] * ilv
                    for j in range(VPR):
                        for k in range(ilv):
                            i = bo + base + tp + k
                            hv = h_v[i, pl.ds(j * LANES, LANES)]
                            rv = r_v[i, pl.ds(j * LANES, LANES)]
                            tv = t_v[i, pl.ds(j * LANES, LANES)]
                            accs[k] = accs[k] + jnp.abs(hv + rv - tv)
                    for k in range(ilv):
                        col = jnp.full((LANES,), tp + k, jnp.int32)
                        plsc.store_scatter(ts_v, [lanes * LANES + col], accs[k])
                vals = [ts_v[pl.ds(l * LANES, LANES)] for l in range(LANES)]
                while len(vals) > 1:
                    vals = [a + b for a, b in zip(vals[::2], vals[1::2])]
                tot = vals[0]
                score = jnp.full((LANES,), GAMMA, jnp.float32) - tot
                obase = jnp.full((LANES,), c * CHUNK, jnp.int32) + base
                plsc.store_scatter(out_v, [obase + lanes], score)
                return carry

            lax.fori_loop(0, CHUNK // LANES, body, 0)

        def out_desc(c):
            return pltpu.make_async_copy(
                out_v.at[pl.ds(c * CHUNK, CHUNK)],
                out_hbm.at[pl.ds(wbase + c * CHUNK, CHUNK)],
                semarr.at[NBUF + 1],
            )

        def chunk_body(c, carry):
            par = lax.rem(c, NBUF)
            bo = par * CHUNK
            sem = semarr.at[par]
            for d in gather_descs(c, bo, sem):
                d.wait()
            compute_chunk(c, bo)

            @pl.when(c + NBUF < n_chunks)
            def _():
                for d in gather_descs(c + NBUF, bo, sem):
                    d.start()

            out_desc(c).start()
            return carry

        lax.fori_loop(0, n_chunks, chunk_body, 0)

        def drain_body(c, carry):
            out_desc(c).wait()
            return carry

        lax.fori_loop(0, n_chunks, drain_body, 0)

    return tec_kernel(entity_emb, relation_emb, head_ids, rel_ids, tail_ids)


# speedup vs baseline: 1.0450x; 1.0257x over previous
import functools

import jax
import jax.numpy as jnp
from jax import lax
from jax.experimental import pallas as pl
from jax.experimental.pallas import tpu as pltpu
from jax.experimental.pallas import tpu_sc as plsc

GAMMA = 12.0
HIDDEN = 128
LANES = 16
VPR = HIDDEN // LANES
CHUNK = 32
NBUF = 8


def kernel(entity_emb, relation_emb, head_ids, rel_ids, tail_ids):
    B = head_ids.shape[0]
    info = plsc.get_sparse_core_info()
    NC, NS = info.num_cores, info.num_subcores
    NW = NC * NS
    per_w = B // NW
    n_chunks = per_w // CHUNK

    mesh = plsc.VectorSubcoreMesh(core_axis_name="c", subcore_axis_name="s")

    @functools.partial(
        pl.kernel,
        mesh=mesh,
        out_type=jax.ShapeDtypeStruct((B,), jnp.float32),
        scratch_types=[
            pltpu.VMEM((per_w,), jnp.int32),
            pltpu.VMEM((per_w,), jnp.int32),
            pltpu.VMEM((per_w,), jnp.int32),
            pltpu.VMEM((NBUF * CHUNK, HIDDEN), jnp.float32),
            pltpu.VMEM((NBUF * CHUNK, HIDDEN), jnp.float32),
            pltpu.VMEM((NBUF * CHUNK, HIDDEN), jnp.float32),
            pltpu.VMEM((per_w,), jnp.float32),
            pltpu.VMEM((LANES * LANES,), jnp.float32),
            pltpu.SemaphoreType.DMA((NBUF + 2,)),
        ],
        compiler_params=pltpu.CompilerParams(needs_layout_passes=False),
    )
    def tec_kernel(ent_hbm, rel_hbm, hid_hbm, rid_hbm, tid_hbm, out_hbm,
                   hi_v, ti_v, ri_v, h_v, t_v, r_v, out_v, ts_v, semarr):
        wid = lax.axis_index("s") * NC + lax.axis_index("c")
        wbase = wid * per_w
        lanes = lax.iota(jnp.int32, LANES)

        wsl = pl.ds(wbase, per_w)
        id_descs = (
            pltpu.make_async_copy(hid_hbm.at[wsl], hi_v, semarr.at[NBUF]),
            pltpu.make_async_copy(tid_hbm.at[wsl], ti_v, semarr.at[NBUF]),
            pltpu.make_async_copy(rid_hbm.at[wsl], ri_v, semarr.at[NBUF]),
        )
        for d in id_descs:
            d.start()
        for d in id_descs:
            d.wait()

        def gather_descs(c, bo, sem):
            sl = pl.ds(c * CHUNK, CHUNK)
            dst = pl.ds(bo, CHUNK)
            return (
                pltpu.make_async_copy(ent_hbm.at[hi_v.at[sl]], h_v.at[dst], sem),
                pltpu.make_async_copy(ent_hbm.at[ti_v.at[sl]], t_v.at[dst], sem),
                pltpu.make_async_copy(rel_hbm.at[ri_v.at[sl]], r_v.at[dst], sem),
            )

        for b in range(NBUF):
            for d in gather_descs(b, b * CHUNK, semarr.at[b]):
                d.start()

        def compute_chunk(c, bo):
            def body(g, carry):
                base = g * LANES
                ilv = 4
                for tp in range(0, LANES, ilv):
                    accs = [jnp.zeros((LANES,), jnp.float32)] * ilv
                    for j in range(VPR):
                        for k in range(ilv):
                            i = bo + base + tp + k
                            hv = h_v[i, pl.ds(j * LANES, LANES)]
                            rv = r_v[i, pl.ds(j * LANES, LANES)]
                            tv = t_v[i, pl.ds(j * LANES, LANES)]
                            accs[k] = accs[k] + jnp.abs(hv + rv - tv)
                    for k in range(ilv):
                        col = jnp.full((LANES,), tp + k, jnp.int32)
                        plsc.store_scatter(ts_v, [lanes * LANES + col], accs[k])
                vals = [ts_v[pl.ds(l * LANES, LANES)] for l in range(LANES)]
                while len(vals) > 1:
                    vals = [a + b for a, b in zip(vals[::2], vals[1::2])]
                tot = vals[0]
                score = jnp.full((LANES,), GAMMA, jnp.float32) - tot
                obase = jnp.full((LANES,), c * CHUNK, jnp.int32) + base
                plsc.store_scatter(out_v, [obase + lanes], score)
                return carry

            lax.fori_loop(0, CHUNK // LANES, body, 0)

        def out_desc(c):
            return pltpu.make_async_copy(
                out_v.at[pl.ds(c * CHUNK, CHUNK)],
                out_hbm.at[pl.ds(wbase + c * CHUNK, CHUNK)],
                semarr.at[NBUF + 1],
            )

        def chunk_body(c, carry):
            par = lax.rem(c, NBUF)
            bo = par * CHUNK
            sem = semarr.at[par]
            for d in gather_descs(c, bo, sem):
                d.wait()
            compute_chunk(c, bo)

            @pl.when(c + NBUF < n_chunks)
            def _():
                for d in gather_descs(c + NBUF, bo, sem):
                    d.start()

            out_desc(c).start()
            return carry

        lax.fori_loop(0, n_chunks, chunk_body, 0)

        def drain_body(c, carry):
            out_desc(c).wait()
            return carry

        lax.fori_loop(0, n_chunks, drain_body, 0)

    return tec_kernel(entity_emb, relation_emb, head_ids, rel_ids, tail_ids)


# speedup vs baseline: 1.0688x; 1.0228x over previous
import functools

import jax
import jax.numpy as jnp
from jax import lax
from jax.experimental import pallas as pl
from jax.experimental.pallas import tpu as pltpu
from jax.experimental.pallas import tpu_sc as plsc

GAMMA = 12.0
HIDDEN = 128
LANES = 16
VPR = HIDDEN // LANES
CHUNK = 16
NBUF = 16


def kernel(entity_emb, relation_emb, head_ids, rel_ids, tail_ids):
    B = head_ids.shape[0]
    info = plsc.get_sparse_core_info()
    NC, NS = info.num_cores, info.num_subcores
    NW = NC * NS
    per_w = B // NW
    n_chunks = per_w // CHUNK

    mesh = plsc.VectorSubcoreMesh(core_axis_name="c", subcore_axis_name="s")

    @functools.partial(
        pl.kernel,
        mesh=mesh,
        out_type=jax.ShapeDtypeStruct((B,), jnp.float32),
        scratch_types=[
            pltpu.VMEM((per_w,), jnp.int32),
            pltpu.VMEM((per_w,), jnp.int32),
            pltpu.VMEM((per_w,), jnp.int32),
            pltpu.VMEM((NBUF * CHUNK, HIDDEN), jnp.float32),
            pltpu.VMEM((NBUF * CHUNK, HIDDEN), jnp.float32),
            pltpu.VMEM((NBUF * CHUNK, HIDDEN), jnp.float32),
            pltpu.VMEM((per_w,), jnp.float32),
            pltpu.VMEM((LANES * LANES,), jnp.float32),
            pltpu.SemaphoreType.DMA((NBUF + 2,)),
        ],
        compiler_params=pltpu.CompilerParams(needs_layout_passes=False),
    )
    def tec_kernel(ent_hbm, rel_hbm, hid_hbm, rid_hbm, tid_hbm, out_hbm,
                   hi_v, ti_v, ri_v, h_v, t_v, r_v, out_v, ts_v, semarr):
        wid = lax.axis_index("s") * NC + lax.axis_index("c")
        wbase = wid * per_w
        lanes = lax.iota(jnp.int32, LANES)

        wsl = pl.ds(wbase, per_w)
        id_descs = (
            pltpu.make_async_copy(hid_hbm.at[wsl], hi_v, semarr.at[NBUF]),
            pltpu.make_async_copy(tid_hbm.at[wsl], ti_v, semarr.at[NBUF]),
            pltpu.make_async_copy(rid_hbm.at[wsl], ri_v, semarr.at[NBUF]),
        )
        for d in id_descs:
            d.start()
        for d in id_descs:
            d.wait()

        def gather_descs(c, bo, sem):
            sl = pl.ds(c * CHUNK, CHUNK)
            dst = pl.ds(bo, CHUNK)
            return (
                pltpu.make_async_copy(ent_hbm.at[hi_v.at[sl]], h_v.at[dst], sem),
                pltpu.make_async_copy(ent_hbm.at[ti_v.at[sl]], t_v.at[dst], sem),
                pltpu.make_async_copy(rel_hbm.at[ri_v.at[sl]], r_v.at[dst], sem),
            )

        for b in range(NBUF):
            for d in gather_descs(b, b * CHUNK, semarr.at[b]):
                d.start()

        def compute_chunk(c, bo):
            def body(g, carry):
                base = g * LANES
                ilv = 4
                for tp in range(0, LANES, ilv):
                    accs = [jnp.zeros((LANES,), jnp.float32)] * ilv
                    for j in range(VPR):
                        for k in range(ilv):
                            i = bo + base + tp + k
                            hv = h_v[i, pl.ds(j * LANES, LANES)]
                            rv = r_v[i, pl.ds(j * LANES, LANES)]
                            tv = t_v[i, pl.ds(j * LANES, LANES)]
                            accs[k] = accs[k] + jnp.abs(hv + rv - tv)
                    for k in range(ilv):
                        col = jnp.full((LANES,), tp + k, jnp.int32)
                        plsc.store_scatter(ts_v, [lanes * LANES + col], accs[k])
                vals = [ts_v[pl.ds(l * LANES, LANES)] for l in range(LANES)]
                while len(vals) > 1:
                    vals = [a + b for a, b in zip(vals[::2], vals[1::2])]
                tot = vals[0]
                score = jnp.full((LANES,), GAMMA, jnp.float32) - tot
                obase = jnp.full((LANES,), c * CHUNK, jnp.int32) + base
                plsc.store_scatter(out_v, [obase + lanes], score)
                return carry

            lax.fori_loop(0, CHUNK // LANES, body, 0)

        def out_desc(c):
            return pltpu.make_async_copy(
                out_v.at[pl.ds(c * CHUNK, CHUNK)],
                out_hbm.at[pl.ds(wbase + c * CHUNK, CHUNK)],
                semarr.at[NBUF + 1],
            )

        def chunk_body(c, carry):
            par = lax.rem(c, NBUF)
            bo = par * CHUNK
            sem = semarr.at[par]
            for d in gather_descs(c, bo, sem):
                d.wait()
            compute_chunk(c, bo)

            @pl.when(c + NBUF < n_chunks)
            def _():
                for d in gather_descs(c + NBUF, bo, sem):
                    d.start()

            out_desc(c).start()
            return carry

        lax.fori_loop(0, n_chunks, chunk_body, 0)

        def drain_body(c, carry):
            out_desc(c).wait()
            return carry

        lax.fori_loop(0, n_chunks, drain_body, 0)

    return tec_kernel(entity_emb, relation_emb, head_ids, rel_ids, tail_ids)


# speedup vs baseline: 1.0722x; 1.0032x over previous
import functools

import jax
import jax.numpy as jnp
from jax import lax
from jax.experimental import pallas as pl
from jax.experimental.pallas import tpu as pltpu
from jax.experimental.pallas import tpu_sc as plsc

GAMMA = 12.0
HIDDEN = 128
LANES = 16
VPR = HIDDEN // LANES
CHUNK = 16
NBUF = 16


def kernel(entity_emb, relation_emb, head_ids, rel_ids, tail_ids):
    B = head_ids.shape[0]
    info = plsc.get_sparse_core_info()
    NC, NS = info.num_cores, info.num_subcores
    NW = NC * NS
    per_w = B // NW
    n_chunks = per_w // CHUNK

    mesh = plsc.VectorSubcoreMesh(core_axis_name="c", subcore_axis_name="s")

    @functools.partial(
        pl.kernel,
        mesh=mesh,
        out_type=jax.ShapeDtypeStruct((B,), jnp.float32),
        scratch_types=[
            pltpu.VMEM((per_w,), jnp.int32),
            pltpu.VMEM((per_w,), jnp.int32),
            pltpu.VMEM((per_w,), jnp.int32),
            pltpu.VMEM((NBUF * CHUNK, HIDDEN), jnp.float32),
            pltpu.VMEM((NBUF * CHUNK, HIDDEN), jnp.float32),
            pltpu.VMEM((NBUF * CHUNK, HIDDEN), jnp.float32),
            pltpu.VMEM((per_w,), jnp.float32),
            pltpu.VMEM((LANES * LANES,), jnp.float32),
            pltpu.SemaphoreType.DMA((NBUF + 2,)),
        ],
        compiler_params=pltpu.CompilerParams(needs_layout_passes=False),
    )
    def tec_kernel(ent_hbm, rel_hbm, hid_hbm, rid_hbm, tid_hbm, out_hbm,
                   hi_v, ti_v, ri_v, h_v, t_v, r_v, out_v, ts_v, semarr):
        wid = lax.axis_index("s") * NC + lax.axis_index("c")
        wbase = wid * per_w
        lanes = lax.iota(jnp.int32, LANES)

        wsl = pl.ds(wbase, per_w)
        id_descs = (
            pltpu.make_async_copy(hid_hbm.at[wsl], hi_v, semarr.at[NBUF]),
            pltpu.make_async_copy(tid_hbm.at[wsl], ti_v, semarr.at[NBUF]),
            pltpu.make_async_copy(rid_hbm.at[wsl], ri_v, semarr.at[NBUF]),
        )
        for d in id_descs:
            d.start()
        for d in id_descs:
            d.wait()

        def gather_descs(c, bo, sem):
            sl = pl.ds(c * CHUNK, CHUNK)
            dst = pl.ds(bo, CHUNK)
            return (
                pltpu.make_async_copy(ent_hbm.at[hi_v.at[sl]], h_v.at[dst], sem),
                pltpu.make_async_copy(ent_hbm.at[ti_v.at[sl]], t_v.at[dst], sem),
                pltpu.make_async_copy(rel_hbm.at[ri_v.at[sl]], r_v.at[dst], sem),
            )

        for b in range(NBUF):
            for d in gather_descs(b, b * CHUNK, semarr.at[b]):
                d.start()

        def compute_chunk(c, bo):
            def body(g, carry):
                base = g * LANES
                ilv = 4
                for tp in range(0, LANES, ilv):
                    accs = [jnp.zeros((LANES,), jnp.float32)] * ilv
                    for j in range(VPR):
                        for k in range(ilv):
                            i = bo + base + tp + k
                            hv = h_v[i, pl.ds(j * LANES, LANES)]
                            rv = r_v[i, pl.ds(j * LANES, LANES)]
                            tv = t_v[i, pl.ds(j * LANES, LANES)]
                            accs[k] = accs[k] + jnp.abs(hv + rv - tv)
                    for k in range(ilv):
                        col = jnp.full((LANES,), tp + k, jnp.int32)
                        plsc.store_scatter(ts_v, [lanes * LANES + col], accs[k])
                vals = [ts_v[pl.ds(l * LANES, LANES)] for l in range(LANES)]
                while len(vals) > 1:
                    vals = [a + b for a, b in zip(vals[::2], vals[1::2])]
                tot = vals[0]
                score = jnp.full((LANES,), GAMMA, jnp.float32) - tot
                obase = jnp.full((LANES,), c * CHUNK, jnp.int32) + base
                plsc.store_scatter(out_v, [obase + lanes], score)
                return carry

            if CHUNK == LANES:
                body(0, 0)
            else:
                lax.fori_loop(0, CHUNK // LANES, body, 0)

        def out_desc(c):
            return pltpu.make_async_copy(
                out_v.at[pl.ds(c * CHUNK, CHUNK)],
                out_hbm.at[pl.ds(wbase + c * CHUNK, CHUNK)],
                semarr.at[NBUF + 1],
            )

        def chunk_body(c, carry):
            par = lax.rem(c, NBUF)
            bo = par * CHUNK
            sem = semarr.at[par]
            for d in gather_descs(c, bo, sem):
                d.wait()
            compute_chunk(c, bo)

            @pl.when(c + NBUF < n_chunks)
            def _():
                for d in gather_descs(c + NBUF, bo, sem):
                    d.start()

            out_desc(c).start()
            return carry

        lax.fori_loop(0, n_chunks, chunk_body, 0)

        def drain_body(c, carry):
            out_desc(c).wait()
            return carry

        lax.fori_loop(0, n_chunks, drain_body, 0)

    return tec_kernel(entity_emb, relation_emb, head_ids, rel_ids, tail_ids)
